# baseline (speedup 1.0000x reference)
import functools

import jax
import jax.numpy as jnp
from jax import lax
from jax.experimental import pallas as pl
from jax.experimental.pallas import tpu as pltpu
from jax.experimental.pallas import tpu_sc as plsc

VOCAB = 1000000
HIDDEN = 32
BATCH = 4096
SEQ = 200


BLK_Y = 2000
VPB = 4 * BLK_Y
NUM_BLOCKS = VOCAB // VPB
GROUP = 32
VPG = GROUP * VPB
NUM_GROUPS = -(-NUM_BLOCKS // GROUP)
T_ROWS = NUM_GROUPS * BLK_Y
T_SIZE = T_ROWS * 128


def _matvec_body(y_ref, s_ref, b_ref, o_ref):
    g = pl.program_id(0)
    j = g % GROUP
    z = jnp.dot(y_ref[...], s_ref[...], preferred_element_type=jnp.float32)
    contrib = pltpu.roll(z, 4 * j, axis=1)

    @pl.when(j == 0)
    def _():
        o_ref[...] = contrib + b_ref[...]

    @pl.when(j != 0)
    def _():
        o_ref[...] += contrib


def _compute_t(table_view, s_mat, b2):
    return pl.pallas_call(
        _matvec_body,
        grid=(NUM_BLOCKS,),
        in_specs=[
            pl.BlockSpec((BLK_Y, 128), lambda i: (i, 0)),
            pl.BlockSpec((128, 128), lambda i: (0, 0)),
            pl.BlockSpec((1, 1), lambda i: (0, 0)),
        ],
        out_specs=pl.BlockSpec((BLK_Y, 128), lambda i: (i // GROUP, 0)),
        out_shape=jax.ShapeDtypeStruct((T_ROWS, 128), jnp.float32),
    )(table_view, s_mat, b2)


NUM_WORKERS = 32
TOK_PER_W = BATCH * SEQ // NUM_WORKERS
SAMP_PER_W = BATCH // NUM_WORKERS
ROWS_PER_W = TOK_PER_W // 128


DEPTH = 16


def _pool_body(ids_hbm, t_hbm, out_hbm, idx_raw, idx_v, vals_v, out_v, sem):
    wid = lax.axis_index("s") * 2 + lax.axis_index("c")
    s0 = wid * SAMP_PER_W
    lane = lax.iota(jnp.int32, 16)

    pltpu.sync_copy(ids_hbm.at[pl.ds(s0, SAMP_PER_W)], idx_raw)

    def xform(i, carry):
        for u in range(13):
            off = u * 16 if u < 12 else SEQ - 16
            v = idx_raw[i, pl.ds(off, 16)]
            big_g = v // VPG
            m = v - big_g * VPG
            jb = m // VPB
            q = m - jb * VPB
            addr = (big_g * VPG
                    + lax.shift_left(lax.shift_right_logical(q, 2), 7)
                    + lax.shift_left(jb, 2)
                    + lax.bitwise_and(q, 3))
            f = i * SEQ + off + lane
            ri = lax.shift_right_logical(f, 7)
            ci = lax.bitwise_and(f, 127)
            if u < 12:
                plsc.store_scatter(idx_v, [ri, ci], addr)
            else:
                plsc.store_scatter(idx_v, [ri, ci], addr, mask=lane >= 8)
        return carry

    lax.fori_loop(0, SAMP_PER_W, xform, 0)

    def fire(r):
        pltpu.async_copy(t_hbm.at[idx_v.at[r]], vals_v.at[r], sem)

    def wait_one():
        pltpu.make_async_copy(t_hbm.at[idx_v.at[0]], vals_v.at[0], sem).wait()

    def prime(r, carry):
        fire(r)
        return carry

    def steady(r, carry):
        fire(r)
        wait_one()
        return carry

    def drain(r, carry):
        wait_one()
        return carry

    lax.fori_loop(0, DEPTH, prime, 0)
    lax.fori_loop(DEPTH, ROWS_PER_W, steady, 0)
    lax.fori_loop(0, DEPTH, drain, 0)

    for g in range(SAMP_PER_W // 16):
        base_f = (g * 16 + lane) * SEQ

        def body(jj, acc):
            for v8 in range(8):
                f = base_f + jj * 8 + v8
                acc = acc + plsc.load_gather(
                    vals_v, [lax.shift_right_logical(f, 7),
                             lax.bitwise_and(f, 127)])
            return acc

        acc = lax.fori_loop(0, SEQ // 8, body, jnp.zeros((16,), jnp.float32))
        out_v[pl.ds(g * 16, 16)] = acc

    pltpu.sync_copy(out_v, out_hbm.at[pl.ds(wid * SAMP_PER_W, SAMP_PER_W)])


@functools.lru_cache(maxsize=1)
def _make_pool():
    return functools.partial(
        pl.kernel,
        mesh=plsc.VectorSubcoreMesh(core_axis_name="c", subcore_axis_name="s"),
        compiler_params=pltpu.CompilerParams(needs_layout_passes=False),
        out_type=jax.ShapeDtypeStruct((BATCH,), jnp.float32),
        scratch_types=[
            pltpu.VMEM((SAMP_PER_W, SEQ), jnp.int32),
            pltpu.VMEM((ROWS_PER_W, 128), jnp.int32),
            pltpu.VMEM((ROWS_PER_W, 128), jnp.float32),
            pltpu.VMEM((SAMP_PER_W,), jnp.float32),
            pltpu.SemaphoreType.DMA,
        ],
    )(_pool_body)


def kernel(input_ids, embed_table, W, b):
    ids = input_ids.astype(jnp.int32)
    table_view = embed_table.reshape(VOCAB // 4, 128)
    k = jnp.arange(128)
    wtile = jnp.tile(W.reshape(HIDDEN).astype(jnp.float32), 4) / SEQ
    s_mat = jnp.where(jnp.arange(128)[None, :] == (k[:, None] // HIDDEN),
                      wtile[:, None], 0.0).astype(jnp.float32)
    b2 = (b.astype(jnp.float32) / SEQ).reshape(1, 1)
    t = _compute_t(table_view, s_mat, b2).reshape(T_SIZE)
    pooled = _make_pool()(ids, t)
    return pooled.reshape(BATCH, 1)


# speedup vs baseline: 9.1243x; 9.1243x over previous
import functools

import jax
import jax.numpy as jnp
from jax import lax
from jax.experimental import pallas as pl
from jax.experimental.pallas import tpu as pltpu
from jax.experimental.pallas import tpu_sc as plsc

VOCAB = 1000000
HIDDEN = 32
BATCH = 4096
SEQ = 200


BN = 65536
NBLK = (VOCAB + BN - 1) // BN


def _matvec_body(x_ref, w_ref, b_ref, o_ref):
    o_ref[...] = jnp.sum(x_ref[...] * w_ref[...], axis=0) + b_ref[0, 0]


def _compute_t(table_t, ws, b2):
    return pl.pallas_call(
        _matvec_body,
        grid=(NBLK,),
        in_specs=[
            pl.BlockSpec((HIDDEN, BN), lambda i: (0, i)),
            pl.BlockSpec((HIDDEN, 1), lambda i: (0, 0)),
            pl.BlockSpec((1, 1), lambda i: (0, 0)),
        ],
        out_specs=pl.BlockSpec((BN,), lambda i: (i,)),
        out_shape=jax.ShapeDtypeStruct((VOCAB,), jnp.float32),
    )(table_t, ws, b2)


NUM_WORKERS = 32
SAMP_PER_W = BATCH // NUM_WORKERS
DEPTH = 16


def _pool_body(ids_hbm, t_hbm, out_hbm, idx_v, vals_v, out_v, sem):
    wid = lax.axis_index("s") * 2 + lax.axis_index("c")
    s0 = wid * SAMP_PER_W

    pltpu.sync_copy(ids_hbm.at[:, pl.ds(s0, SAMP_PER_W)], idx_v)

    def fire(r):
        pltpu.async_copy(t_hbm.at[idx_v.at[r]], vals_v.at[r], sem)

    def wait_one():
        pltpu.make_async_copy(t_hbm.at[idx_v.at[0]], vals_v.at[0], sem).wait()

    def prime(r, carry):
        fire(r)
        return carry

    def steady(r, carry):
        fire(r)
        wait_one()
        return carry

    def drain(r, carry):
        wait_one()
        return carry

    lax.fori_loop(0, DEPTH, prime, 0)
    lax.fori_loop(DEPTH, SEQ, steady, 0)
    lax.fori_loop(0, DEPTH, drain, 0)

    for g in range(SAMP_PER_W // 16):

        def body(j, acc):
            for u in range(4):
                acc = acc + vals_v[j * 4 + u, pl.ds(g * 16, 16)]
            return acc

        acc = lax.fori_loop(0, SEQ // 4, body, jnp.zeros((16,), jnp.float32))
        out_v[pl.ds(g * 16, 16)] = acc

    pltpu.sync_copy(out_v, out_hbm.at[pl.ds(s0, SAMP_PER_W)])


@functools.lru_cache(maxsize=1)
def _make_pool():
    return functools.partial(
        pl.kernel,
        mesh=plsc.VectorSubcoreMesh(core_axis_name="c", subcore_axis_name="s"),
        compiler_params=pltpu.CompilerParams(needs_layout_passes=False),
        out_type=jax.ShapeDtypeStruct((BATCH,), jnp.float32),
        scratch_types=[
            pltpu.VMEM((SEQ, 128), jnp.int32),
            pltpu.VMEM((SEQ, 128), jnp.float32),
            pltpu.VMEM((SAMP_PER_W,), jnp.float32),
            pltpu.SemaphoreType.DMA,
        ],
    )(_pool_body)


def kernel(input_ids, embed_table, W, b):
    ids_t = input_ids.astype(jnp.int32).T
    table_t = embed_table.T
    ws = W.astype(jnp.float32) / SEQ
    b2 = (b.astype(jnp.float32) / SEQ).reshape(1, 1)
    t = _compute_t(table_t, ws, b2)
    pooled = _make_pool()(ids_t, t)
    return pooled.reshape(BATCH, 1)


# speedup vs baseline: 9.4564x; 1.0364x over previous
import functools

import jax
import jax.numpy as jnp
from jax import lax
from jax.experimental import pallas as pl
from jax.experimental.pallas import tpu as pltpu
from jax.experimental.pallas import tpu_sc as plsc

VOCAB = 1000000
HIDDEN = 32
BATCH = 4096
SEQ = 200


BN = 65536
NBLK = (VOCAB + BN - 1) // BN


def _matvec_body(x_ref, w_ref, b_ref, o_ref):
    o_ref[...] = jnp.sum(x_ref[...] * w_ref[...], axis=0) + b_ref[0, 0]


def _compute_t(table_t, ws, b2):
    return pl.pallas_call(
        _matvec_body,
        grid=(NBLK,),
        in_specs=[
            pl.BlockSpec((HIDDEN, BN), lambda i: (0, i)),
            pl.BlockSpec((HIDDEN, 1), lambda i: (0, 0)),
            pl.BlockSpec((1, 1), lambda i: (0, 0)),
        ],
        out_specs=pl.BlockSpec((BN,), lambda i: (i,)),
        out_shape=jax.ShapeDtypeStruct((VOCAB,), jnp.float32),
    )(table_t, ws, b2)


NUM_WORKERS = 32
SAMP_PER_W = BATCH // NUM_WORKERS
GB = 40
NBATCH = SEQ // GB
NGRP = SAMP_PER_W // 16


def _pool_body(ids_hbm, t_hbm, out_hbm, idx_v, vals_v, out_v, sem0, sem1):
    wid = lax.axis_index("s") * 2 + lax.axis_index("c")
    s0 = wid * SAMP_PER_W
    sems = (sem0, sem1)

    pltpu.sync_copy(ids_hbm.at[:, pl.ds(s0, SAMP_PER_W)], idx_v)

    def fire_batch(k):
        for r in range(k * GB, (k + 1) * GB):
            pltpu.async_copy(t_hbm.at[idx_v.at[r]], vals_v.at[r], sems[k % 2])

    def drain_batch(k):
        for _ in range(GB):
            pltpu.make_async_copy(
                t_hbm.at[idx_v.at[0]], vals_v.at[0], sems[k % 2]).wait()

    accs = [jnp.zeros((16,), jnp.float32) for _ in range(NGRP)]

    def accum_batch(k, accs):
        out = list(accs)
        for r in range(k * GB, (k + 1) * GB):
            for g in range(NGRP):
                out[g] = out[g] + vals_v[r, pl.ds(g * 16, 16)]
        return out

    fire_batch(0)
    for k in range(1, NBATCH):
        fire_batch(k)
        drain_batch(k - 1)
        accs = accum_batch(k - 1, accs)
    drain_batch(NBATCH - 1)
    accs = accum_batch(NBATCH - 1, accs)

    for g in range(NGRP):
        out_v[pl.ds(g * 16, 16)] = accs[g]

    pltpu.sync_copy(out_v, out_hbm.at[pl.ds(s0, SAMP_PER_W)])


@functools.lru_cache(maxsize=1)
def _make_pool():
    return functools.partial(
        pl.kernel,
        mesh=plsc.VectorSubcoreMesh(core_axis_name="c", subcore_axis_name="s"),
        compiler_params=pltpu.CompilerParams(needs_layout_passes=False),
        out_type=jax.ShapeDtypeStruct((BATCH,), jnp.float32),
        scratch_types=[
            pltpu.VMEM((SEQ, 128), jnp.int32),
            pltpu.VMEM((SEQ, 128), jnp.float32),
            pltpu.VMEM((SAMP_PER_W,), jnp.float32),
            pltpu.SemaphoreType.DMA,
            pltpu.SemaphoreType.DMA,
        ],
    )(_pool_body)


def kernel(input_ids, embed_table, W, b):
    ids_t = input_ids.astype(jnp.int32).T
    table_t = embed_table.T
    ws = W.astype(jnp.float32) / SEQ
    b2 = (b.astype(jnp.float32) / SEQ).reshape(1, 1)
    t = _compute_t(table_t, ws, b2)
    pooled = _make_pool()(ids_t, t)
    return pooled.reshape(BATCH, 1)


# speedup vs baseline: 9.5536x; 1.0103x over previous
import functools

import jax
import jax.numpy as jnp
from jax import lax
from jax.experimental import pallas as pl
from jax.experimental.pallas import tpu as pltpu
from jax.experimental.pallas import tpu_sc as plsc

VOCAB = 1000000
HIDDEN = 32
BATCH = 4096
SEQ = 200


BN = 131072
NBLK = (VOCAB + BN - 1) // BN


def _matvec_body(x_ref, w_ref, b_ref, o_ref):
    o_ref[...] = jnp.sum(x_ref[...] * w_ref[...], axis=0) + b_ref[0, 0]


def _compute_t(table_t, ws, b2):
    return pl.pallas_call(
        _matvec_body,
        grid=(NBLK,),
        in_specs=[
            pl.BlockSpec((HIDDEN, BN), lambda i: (0, i)),
            pl.BlockSpec((HIDDEN, 1), lambda i: (0, 0)),
            pl.BlockSpec((1, 1), lambda i: (0, 0)),
        ],
        out_specs=pl.BlockSpec((BN,), lambda i: (i,)),
        out_shape=jax.ShapeDtypeStruct((VOCAB,), jnp.float32),
    )(table_t, ws, b2)


NUM_WORKERS = 32
SAMP_PER_W = BATCH // NUM_WORKERS
GB = 40
NBATCH = SEQ // GB
NGRP = SAMP_PER_W // 16


def _pool_body(ids_hbm, t_hbm, out_hbm, idx_v, vals_v, out_v, sem0, sem1):
    wid = lax.axis_index("s") * 2 + lax.axis_index("c")
    s0 = wid * SAMP_PER_W
    sems = (sem0, sem1)

    pltpu.sync_copy(ids_hbm.at[:, pl.ds(s0, SAMP_PER_W)], idx_v)

    def fire_batch(k):
        for r in range(k * GB, (k + 1) * GB):
            pltpu.async_copy(t_hbm.at[idx_v.at[r]], vals_v.at[r], sems[k % 2])

    def drain_batch(k):
        for _ in range(GB):
            pltpu.make_async_copy(
                t_hbm.at[idx_v.at[0]], vals_v.at[0], sems[k % 2]).wait()

    accs = [jnp.zeros((16,), jnp.float32) for _ in range(NGRP)]

    def accum_batch(k, accs):
        out = list(accs)
        for r in range(k * GB, (k + 1) * GB):
            for g in range(NGRP):
                out[g] = out[g] + vals_v[r, pl.ds(g * 16, 16)]
        return out

    fire_batch(0)
    for k in range(1, NBATCH):
        fire_batch(k)
        drain_batch(k - 1)
        accs = accum_batch(k - 1, accs)
    drain_batch(NBATCH - 1)
    accs = accum_batch(NBATCH - 1, accs)

    for g in range(NGRP):
        out_v[pl.ds(g * 16, 16)] = accs[g]

    pltpu.sync_copy(out_v, out_hbm.at[pl.ds(s0, SAMP_PER_W)])


@functools.lru_cache(maxsize=1)
def _make_pool():
    return functools.partial(
        pl.kernel,
        mesh=plsc.VectorSubcoreMesh(core_axis_name="c", subcore_axis_name="s"),
        compiler_params=pltpu.CompilerParams(needs_layout_passes=False),
        out_type=jax.ShapeDtypeStruct((BATCH,), jnp.float32),
        scratch_types=[
            pltpu.VMEM((SEQ, 128), jnp.int32),
            pltpu.VMEM((SEQ, 128), jnp.float32),
            pltpu.VMEM((SAMP_PER_W,), jnp.float32),
            pltpu.SemaphoreType.DMA,
            pltpu.SemaphoreType.DMA,
        ],
    )(_pool_body)


def kernel(input_ids, embed_table, W, b):
    ids_t = input_ids.astype(jnp.int32).T
    table_t = embed_table.T
    ws = W.astype(jnp.float32) / SEQ
    b2 = (b.astype(jnp.float32) / SEQ).reshape(1, 1)
    t = _compute_t(table_t, ws, b2)
    pooled = _make_pool()(ids_t, t)
    return pooled.reshape(BATCH, 1)


# speedup vs baseline: 9.5835x; 1.0031x over previous
import functools

import jax
import jax.numpy as jnp
from jax import lax
from jax.experimental import pallas as pl
from jax.experimental.pallas import tpu as pltpu
from jax.experimental.pallas import tpu_sc as plsc

VOCAB = 1000000
HIDDEN = 32
BATCH = 4096
SEQ = 200


BN = 131072
NBLK = (VOCAB + BN - 1) // BN


def _matvec_body(x_ref, w_ref, b_ref, o_ref):
    o_ref[...] = jnp.sum(x_ref[...] * w_ref[...], axis=0) + b_ref[0, 0]


def _compute_t(table_t, ws, b2):
    return pl.pallas_call(
        _matvec_body,
        grid=(NBLK,),
        in_specs=[
            pl.BlockSpec((HIDDEN, BN), lambda i: (0, i)),
            pl.BlockSpec((HIDDEN, 1), lambda i: (0, 0)),
            pl.BlockSpec((1, 1), lambda i: (0, 0)),
        ],
        out_specs=pl.BlockSpec((BN,), lambda i: (i,)),
        out_shape=jax.ShapeDtypeStruct((VOCAB,), jnp.float32),
    )(table_t, ws, b2)


NUM_WORKERS = 32
SAMP_PER_W = BATCH // NUM_WORKERS
GB = 40
NBATCH = SEQ // GB
NGRP = SAMP_PER_W // 16


def _pool_body(ids_hbm, t_hbm, out_hbm, idx_v, vals_v, out_v, sem0, sem1):
    wid = lax.axis_index("s") * 2 + lax.axis_index("c")
    s0 = wid * SAMP_PER_W
    sems = (sem0, sem1)

    pltpu.sync_copy(ids_hbm.at[pl.ds(0, GB), pl.ds(s0, SAMP_PER_W)],
                    idx_v.at[pl.ds(0, GB)])

    def fire_batch(k):
        for r in range(k * GB, (k + 1) * GB):
            pltpu.async_copy(t_hbm.at[idx_v.at[r]], vals_v.at[r], sems[k % 2])

    def drain_batch(k):
        for _ in range(GB):
            pltpu.make_async_copy(
                t_hbm.at[idx_v.at[0]], vals_v.at[0], sems[k % 2]).wait()

    accs = [jnp.zeros((16,), jnp.float32) for _ in range(NGRP)]

    def accum_batch(k, accs):
        out = list(accs)
        for r in range(k * GB, (k + 1) * GB):
            for g in range(NGRP):
                out[g] = out[g] + vals_v[r, pl.ds(g * 16, 16)]
        return out

    fire_batch(0)
    pltpu.sync_copy(ids_hbm.at[pl.ds(GB, SEQ - GB), pl.ds(s0, SAMP_PER_W)],
                    idx_v.at[pl.ds(GB, SEQ - GB)])
    for k in range(1, NBATCH):
        fire_batch(k)
        drain_batch(k - 1)
        accs = accum_batch(k - 1, accs)
    drain_batch(NBATCH - 1)
    accs = accum_batch(NBATCH - 1, accs)

    for g in range(NGRP):
        out_v[pl.ds(g * 16, 16)] = accs[g]

    pltpu.sync_copy(out_v, out_hbm.at[pl.ds(s0, SAMP_PER_W)])


@functools.lru_cache(maxsize=1)
def _make_pool():
    return functools.partial(
        pl.kernel,
        mesh=plsc.VectorSubcoreMesh(core_axis_name="c", subcore_axis_name="s"),
        compiler_params=pltpu.CompilerParams(needs_layout_passes=False),
        out_type=jax.ShapeDtypeStruct((BATCH,), jnp.float32),
        scratch_types=[
            pltpu.VMEM((SEQ, 128), jnp.int32),
            pltpu.VMEM((SEQ, 128), jnp.float32),
            pltpu.VMEM((SAMP_PER_W,), jnp.float32),
            pltpu.SemaphoreType.DMA,
            pltpu.SemaphoreType.DMA,
        ],
    )(_pool_body)


def kernel(input_ids, embed_table, W, b):
    ids_t = input_ids.astype(jnp.int32).T
    table_t = embed_table.T
    ws = W.astype(jnp.float32) / SEQ
    b2 = (b.astype(jnp.float32) / SEQ).reshape(1, 1)
    t = _compute_t(table_t, ws, b2)
    pooled = _make_pool()(ids_t, t)
    return pooled.reshape(BATCH, 1)


# speedup vs baseline: 9.6576x; 1.0077x over previous
import functools

import jax
import jax.numpy as jnp
from jax import lax
from jax.experimental import pallas as pl
from jax.experimental.pallas import tpu as pltpu
from jax.experimental.pallas import tpu_sc as plsc

VOCAB = 1000000
HIDDEN = 32
BATCH = 4096
SEQ = 200


BN = 131072
NBLK = (VOCAB + BN - 1) // BN


def _matvec_body(x_ref, w_ref, b_ref, o_ref):
    o_ref[...] = jnp.sum(x_ref[...] * w_ref[...], axis=0) + b_ref[0, 0]


def _compute_t(table_t, ws, b2):
    return pl.pallas_call(
        _matvec_body,
        grid=(NBLK,),
        in_specs=[
            pl.BlockSpec((HIDDEN, BN), lambda i: (0, i)),
            pl.BlockSpec((HIDDEN, 1), lambda i: (0, 0)),
            pl.BlockSpec((1, 1), lambda i: (0, 0)),
        ],
        out_specs=pl.BlockSpec((BN,), lambda i: (i,)),
        out_shape=jax.ShapeDtypeStruct((VOCAB,), jnp.float32),
    )(table_t, ws, b2)


NUM_WORKERS = 32
SAMP_PER_W = BATCH // NUM_WORKERS
GB = 25
NBATCH = SEQ // GB
NSEM = 3
NGRP = SAMP_PER_W // 16


def _pool_body(ids_hbm, t_hbm, out_hbm, idx_v, vals_v, out_v,
               sem0, sem1, sem2):
    wid = lax.axis_index("s") * 2 + lax.axis_index("c")
    s0 = wid * SAMP_PER_W
    sems = (sem0, sem1, sem2)

    SPLIT = 80
    pltpu.sync_copy(ids_hbm.at[pl.ds(0, SPLIT), pl.ds(s0, SAMP_PER_W)],
                    idx_v.at[pl.ds(0, SPLIT)])

    def fire_batch(k):
        for r in range(k * GB, (k + 1) * GB):
            pltpu.async_copy(t_hbm.at[idx_v.at[r]], vals_v.at[r],
                             sems[k % NSEM])

    def drain_batch(k):
        for _ in range(GB):
            pltpu.make_async_copy(
                t_hbm.at[idx_v.at[0]], vals_v.at[0], sems[k % NSEM]).wait()

    accs = [jnp.zeros((16,), jnp.float32) for _ in range(NGRP)]

    def accum_batch(k, accs):
        out = list(accs)
        for r in range(k * GB, (k + 1) * GB):
            for g in range(NGRP):
                out[g] = out[g] + vals_v[r, pl.ds(g * 16, 16)]
        return out

    for k in range(NSEM):
        fire_batch(k)
    pltpu.sync_copy(
        ids_hbm.at[pl.ds(SPLIT, SEQ - SPLIT), pl.ds(s0, SAMP_PER_W)],
        idx_v.at[pl.ds(SPLIT, SEQ - SPLIT)])
    for k in range(NSEM, NBATCH):
        drain_batch(k - NSEM)
        fire_batch(k)
        accs = accum_batch(k - NSEM, accs)
    for k in range(NBATCH - NSEM, NBATCH):
        drain_batch(k)
        accs = accum_batch(k, accs)

    for g in range(NGRP):
        out_v[pl.ds(g * 16, 16)] = accs[g]

    pltpu.sync_copy(out_v, out_hbm.at[pl.ds(s0, SAMP_PER_W)])


@functools.lru_cache(maxsize=1)
def _make_pool():
    return functools.partial(
        pl.kernel,
        mesh=plsc.VectorSubcoreMesh(core_axis_name="c", subcore_axis_name="s"),
        compiler_params=pltpu.CompilerParams(needs_layout_passes=False),
        out_type=jax.ShapeDtypeStruct((BATCH,), jnp.float32),
        scratch_types=[
            pltpu.VMEM((SEQ, 128), jnp.int32),
            pltpu.VMEM((SEQ, 128), jnp.float32),
            pltpu.VMEM((SAMP_PER_W,), jnp.float32),
            pltpu.SemaphoreType.DMA,
            pltpu.SemaphoreType.DMA,
            pltpu.SemaphoreType.DMA,
        ],
    )(_pool_body)


def kernel(input_ids, embed_table, W, b):
    ids_t = input_ids.astype(jnp.int32).T
    table_t = embed_table.T
    ws = W.astype(jnp.float32) / SEQ
    b2 = (b.astype(jnp.float32) / SEQ).reshape(1, 1)
    t = _compute_t(table_t, ws, b2)
    pooled = _make_pool()(ids_t, t)
    return pooled.reshape(BATCH, 1)
